# baseline (speedup 1.0000x reference)
import jax
import jax.numpy as jnp
from jax import lax
from jax.experimental import pallas as pl
from jax.experimental.pallas import tpu as pltpu
from jax.experimental.pallas import tpu_sc as plsc

_NUM_PRE = 100000
_EMBED_DIM = 64
_BATCH = 16384

_L = 16
_NS = 16
_HALF = _BATCH // 2
_BPW = _HALF // _NS
_C = 16
_NBUF = _BPW + 2 * _C
_DUMP = _HALF


def _body(idx_hbm, w_pre_hbm, w_new_hbm, out_hbm,
          idx_stage, idx_buf, pos_buf, pos2d, rows_v, sem, sem2):
    wid = lax.axis_index("s")
    base = wid * _BPW
    lane = lax.iota(jnp.int32, _L)

    pltpu.sync_copy(idx_hbm.at[pl.ds(base, _BPW)], idx_stage)

    def init_body(i, _):
        idx_buf[pl.ds(i * _L, _L)] = jnp.zeros((_L,), jnp.int32)
        pos_buf[pl.ds(i * _L, _L)] = jnp.full((_L,), _DUMP, jnp.int32)
        return 0
    lax.fori_loop(0, _NBUF // _L, init_body, 0)

    def count_body(r, cntv):
        v = idx_stage[pl.ds(r * _L, _L)]
        return cntv + jnp.where(v < _NUM_PRE, 1, 0).astype(jnp.int32)
    cnt_v = lax.fori_loop(0, _BPW // _L, count_body,
                          jnp.zeros((_L,), jnp.int32))
    n_lo = jnp.sum(cnt_v)
    n_hi = _BPW - n_lo
    n_lo_pad = ((n_lo + _C - 1) // _C) * _C

    def compact_body(r, carry):
        o_lo, o_hi = carry
        v = idx_stage[pl.ds(r * _L, _L)]
        pos = base + r * _L + lane
        m_lo = v < _NUM_PRE
        m_hi = jnp.logical_not(m_lo)
        plsc.store_compressed(idx_buf.at[pl.ds(o_lo, _L)], v, mask=m_lo)
        plsc.store_compressed(pos_buf.at[pl.ds(o_lo, _L)], pos, mask=m_lo)
        plsc.store_compressed(idx_buf.at[pl.ds(o_hi, _L)], v - _NUM_PRE,
                              mask=m_hi)
        plsc.store_compressed(pos_buf.at[pl.ds(o_hi, _L)], pos, mask=m_hi)
        c = jnp.sum(jnp.where(m_lo, 1, 0).astype(jnp.int32))
        return o_lo + c, o_hi + (_L - c)
    lax.fori_loop(0, _BPW // _L, compact_body, (jnp.int32(0), n_lo_pad))

    n_lo_ch = n_lo_pad // _C
    n_hi_ch = (n_hi + _C - 1) // _C

    def fire_lo(j, _):
        pltpu.async_copy(w_pre_hbm.at[idx_buf.at[pl.ds(j * _C, _C)]],
                         rows_v.at[pl.ds(j * _C, _C), :], sem)
        return 0
    lax.fori_loop(0, n_lo_ch, fire_lo, 0)

    def fire_hi(j, _):
        off = n_lo_pad + j * _C
        pltpu.async_copy(w_new_hbm.at[idx_buf.at[pl.ds(off, _C)]],
                         rows_v.at[pl.ds(off, _C), :], sem)
        return 0
    lax.fori_loop(0, n_hi_ch, fire_hi, 0)

    def drain(j, _):
        pltpu.make_async_copy(w_pre_hbm.at[idx_buf.at[pl.ds(0, _C)]],
                              rows_v.at[pl.ds(0, _C), :], sem).wait()
        return 0
    lax.fori_loop(0, n_lo_ch + n_hi_ch, drain, 0)

    def pos_copy(j, _):
        pos2d[j, :] = pos_buf[pl.ds(j * _C, _C)]
        return 0
    lax.fori_loop(0, _NBUF // _C, pos_copy, 0)

    def fire_scatter(j, _):
        pltpu.async_copy(rows_v.at[pl.ds(j * _C, _C), :],
                         out_hbm.at[pos2d.at[j]], sem2)
        return 0
    lax.fori_loop(0, _NBUF // _C, fire_scatter, 0)

    def drain_scatter(j, _):
        pltpu.make_async_copy(rows_v.at[pl.ds(0, _C), :],
                              out_hbm.at[pos2d.at[0]], sem2).wait()
        return 0
    lax.fori_loop(0, _NBUF // _C, drain_scatter, 0)


def _make_half():
    return pl.kernel(
        _body,
        out_type=jax.ShapeDtypeStruct((_HALF + 1, _EMBED_DIM), jnp.float32),
        mesh=plsc.VectorSubcoreMesh(core_axis_name="c", subcore_axis_name="s",
                                    num_cores=1, num_subcores=_NS),
        scratch_types=[
            pltpu.VMEM((_BPW,), jnp.int32),
            pltpu.VMEM((_NBUF,), jnp.int32),
            pltpu.VMEM((_NBUF,), jnp.int32),
            pltpu.VMEM((_NBUF // _C, _C), jnp.int32),
            pltpu.VMEM((_NBUF, _EMBED_DIM), jnp.float32),
            pltpu.SemaphoreType.DMA,
            pltpu.SemaphoreType.DMA,
        ],
        compiler_params=pltpu.CompilerParams(use_tc_tiling_on_sc=False,
                                             needs_layout_passes=False),
    )


@jax.jit
def _combined_lookup(indices, w_pre, w_new):
    run = _make_half()
    o0 = run(indices[:_HALF], w_pre, w_new)
    o1 = run(indices[_HALF:], w_pre, w_new)
    return jnp.concatenate([o0[:_HALF], o1[:_HALF]], axis=0)


def kernel(indices, W_pre, W_new):
    return _combined_lookup(indices.astype(jnp.int32), W_pre, W_new)


# speedup vs baseline: 1.4863x; 1.4863x over previous
import jax
import jax.numpy as jnp
from jax import lax
from jax.experimental import pallas as pl
from jax.experimental.pallas import tpu as pltpu
from jax.experimental.pallas import tpu_sc as plsc

_NUM_PRE = 100000
_EMBED_DIM = 64
_BATCH = 16384

_L = 16
_NC, _NS = 2, 16
_NW = _NC * _NS
_BW = 128
_NBLK1 = (_NUM_PRE + _BW - 1) // _BW
_PARTIAL_W = _NUM_PRE - (_NBLK1 - 1) * _BW
_NBLK = 2 * _NBLK1
_BPW = (_NBLK + _NW - 1) // _NW
_NVR = _BATCH // _L
_RING = 8
_OUTW = 128


def _body(idx_hbm, w_pre_hbm, w_new_hbm, out_hbm,
          idxs, s_vi, s_pos, hist, off, bbuf, stage, posr,
          sem_blk, sem_sc):
    wid = lax.axis_index("s") * _NC + lax.axis_index("c")
    b0 = wid * _BPW
    b1 = jnp.minimum(b0 + _BPW, _NBLK)
    nb = b1 - b0
    dump = _BATCH + wid
    lane = lax.iota(jnp.int32, _L)
    ones = jnp.ones((_L,), jnp.int32)

    pltpu.sync_copy(idx_hbm, idxs)

    for i in range(4):
        hist[pl.ds(i * _L, _L)] = jnp.zeros((_L,), jnp.int32)

    def _classify(r):
        v = idxs[pl.ds(r * _L, _L)]
        m_pre = v < _NUM_PRE
        vloc = jnp.where(m_pre, v, v - _NUM_PRE)
        blk = jnp.where(m_pre, 0, _NBLK1) + (vloc >> 7)
        mine = jnp.logical_and(blk >= b0, blk < b1)
        brel = jnp.where(mine, blk - b0, 0)
        return vloc, brel, mine

    def scan1(r, _):
        _, brel, mine = _classify(r)
        plsc.addupdate_scatter(hist, [brel], ones, mask=mine)
        return 0
    lax.fori_loop(0, _NVR, scan1, 0)

    carry = jnp.int32(0)
    for i in range(4):
        h = hist[pl.ds(i * _L, _L)]
        off[pl.ds(i * _L, _L)] = plsc.cumsum(h) - h + carry
        carry = carry + jnp.sum(h)

    def scan2(r, _):
        vloc, brel, mine = _classify(r)
        occ, _last = plsc.scan_count(brel, mask=mine)
        g = plsc.load_gather(off, [brel], mask=mine)
        slot = g + occ - 1
        plsc.store_scatter(s_vi, [slot], vloc & (_BW - 1), mask=mine)
        plsc.store_scatter(s_pos, [slot], r * _L + lane, mask=mine)
        plsc.addupdate_scatter(off, [brel], ones, mask=mine)
        return 0
    lax.fori_loop(0, _NVR, scan2, 0)

    def _fire(b, buf):
        is_new = b >= _NBLK1
        bt = jnp.where(is_new, b - _NBLK1, b)
        col = bt * _BW

        def full_pre():
            pltpu.async_copy(w_pre_hbm.at[:, pl.ds(col, _BW)],
                             bbuf.at[buf], sem_blk)
        def full_new():
            pltpu.async_copy(w_new_hbm.at[:, pl.ds(col, _BW)],
                             bbuf.at[buf], sem_blk)

        lax.cond(is_new, full_new, full_pre)

    def _wait_block(b):
        pltpu.make_async_copy(w_pre_hbm.at[:, pl.ds(0, _BW)],
                              bbuf.at[0], sem_blk).wait()

    @pl.when(nb > 0)
    def _():
        _fire(b0, 0)

    def bloop(j, carry):
        start, gc = carry
        b = b0 + j
        _wait_block(b)

        @pl.when(j + 1 < nb)
        def _():
            _fire(b + 1, (j + 1) & 1)

        cnt = jnp.int32(0)
        for i in range(4):
            h = hist[pl.ds(i * _L, _L)]
            cnt = cnt + jnp.sum(jnp.where(lane + i * _L == j, h, 0))

        buf = j & 1

        def gloop(g, gc):
            s0 = start + g * _L
            raw_vi = s_vi[pl.ds(s0, _L)]
            raw_pos = s_pos[pl.ds(s0, _L)]
            valid = g * _L + lane < cnt
            vi = jnp.where(valid, raw_vi, 0)
            pos = jnp.where(valid, raw_pos, dump)
            rs = gc & (_RING - 1)

            @pl.when(gc >= _RING)
            def _():
                pltpu.make_async_copy(stage.at[0],
                                      out_hbm.at[posr.at[0]],
                                      sem_sc.at[rs]).wait()

            bvec = jnp.full((_L,), buf, jnp.int32)
            rvec = jnp.full((_L,), rs, jnp.int32)
            for c in range(_EMBED_DIM):
                cvec = jnp.full((_L,), c, jnp.int32)
                x = plsc.load_gather(bbuf, [bvec, cvec, vi])
                plsc.store_scatter(stage, [rvec, lane, cvec], x)
            plsc.store_scatter(posr, [rvec, lane], pos)
            pltpu.async_copy(stage.at[rs], out_hbm.at[posr.at[rs]],
                             sem_sc.at[rs])
            return gc + 1

        n_g = (cnt + _L - 1) // _L
        gc = lax.fori_loop(0, n_g, gloop, gc)
        return start + cnt, gc

    _, gc_end = lax.fori_loop(0, nb, bloop, (jnp.int32(0), jnp.int32(0)))

    def drain(k, _):
        @pl.when(k < jnp.minimum(gc_end, _RING))
        def _():
            pltpu.make_async_copy(stage.at[0], out_hbm.at[posr.at[0]],
                                  sem_sc.at[k]).wait()
        return 0
    lax.fori_loop(0, _RING, drain, 0)


@jax.jit
def _combined_lookup(indices, w_pre, w_new):
    run = pl.kernel(
        _body,
        out_type=jax.ShapeDtypeStruct((_BATCH + _NW, _OUTW), jnp.float32),
        mesh=plsc.VectorSubcoreMesh(core_axis_name="c", subcore_axis_name="s",
                                    num_cores=_NC, num_subcores=_NS),
        scratch_types=[
            pltpu.VMEM((_BATCH,), jnp.int32),
            pltpu.VMEM((_BATCH,), jnp.int32),
            pltpu.VMEM((_BATCH,), jnp.int32),
            pltpu.VMEM((64,), jnp.int32),
            pltpu.VMEM((64,), jnp.int32),
            pltpu.VMEM((2, _EMBED_DIM, _BW), jnp.float32),
            pltpu.VMEM((_RING, _L, _OUTW), jnp.float32),
            pltpu.VMEM((_RING, _L), jnp.int32),
            pltpu.SemaphoreType.DMA,
            pltpu.SemaphoreType.DMA((_RING,)),
        ],
        compiler_params=pltpu.CompilerParams(use_tc_tiling_on_sc=True,
                                             needs_layout_passes=False),
    )
    return run(indices, w_pre.T, w_new.T)


def kernel(indices, W_pre, W_new):
    out = _combined_lookup(indices.astype(jnp.int32), W_pre, W_new)
    return out[:_BATCH, :_EMBED_DIM]
